# Initial kernel scaffold; baseline (speedup 1.0000x reference)
import jax
import jax.numpy as jnp
from jax.experimental import pallas as pl


def kernel(y_cls, y_cli, e_cls, e_cli):
    raise NotImplementedError("write your pallas kernel here")


# speedup vs baseline: 1.4543x; 1.4543x over previous
import functools

import jax
import jax.numpy as jnp
from jax import lax
from jax.experimental import pallas as pl
from jax.experimental.pallas import tpu as pltpu
from jax.experimental.pallas import tpu_sc as plsc


@functools.lru_cache(maxsize=None)
def _build(B, D1, D2):
    info = plsc.get_sparse_core_info()
    num_workers = info.num_cores * info.num_subcores
    assert B % num_workers == 0
    bpw = B // num_workers
    mesh = plsc.VectorSubcoreMesh(core_axis_name="c", subcore_axis_name="s")

    @functools.partial(
        pl.kernel,
        mesh=mesh,
        compiler_params=pltpu.CompilerParams(use_tc_tiling_on_sc=False),
        out_type=jax.ShapeDtypeStruct((B, D1 + D2), jnp.float32),
        scratch_types=[
            pltpu.VMEM((bpw,), jnp.int32),
            pltpu.VMEM((bpw,), jnp.int32),
            pltpu.VMEM((bpw, D1), jnp.float32),
            pltpu.VMEM((bpw, D2), jnp.float32),
            pltpu.SemaphoreType.DMA,
            pltpu.SemaphoreType.DMA,
        ],
    )
    def cond_embed(y_cls_hbm, y_cli_hbm, e_cls_hbm, e_cli_hbm, out_hbm,
                   idx1_v, idx2_v, rows1_v, rows2_v, sem1, sem2):
        wid = lax.axis_index("s") * info.num_cores + lax.axis_index("c")
        base = wid * bpw
        pltpu.sync_copy(y_cls_hbm.at[pl.ds(base, bpw)], idx1_v)
        pltpu.sync_copy(y_cli_hbm.at[pl.ds(base, bpw)], idx2_v)
        g1 = pltpu.async_copy(e_cls_hbm.at[idx1_v], rows1_v, sem1)
        g2 = pltpu.async_copy(e_cli_hbm.at[idx2_v], rows2_v, sem2)
        g1.wait()
        pltpu.sync_copy(rows1_v, out_hbm.at[pl.ds(base, bpw), pl.ds(0, D1)])
        g2.wait()
        pltpu.sync_copy(rows2_v, out_hbm.at[pl.ds(base, bpw), pl.ds(D1, D2)])

    return cond_embed


def kernel(y_cls, y_cli, e_cls, e_cli):
    f = _build(y_cls.shape[0], e_cls.shape[1], e_cli.shape[1])
    return f(y_cls.astype(jnp.int32), y_cli.astype(jnp.int32), e_cls, e_cli)


# speedup vs baseline: 1.5079x; 1.0369x over previous
import functools

import jax
import jax.numpy as jnp
from jax import lax
from jax.experimental import pallas as pl
from jax.experimental.pallas import tpu as pltpu
from jax.experimental.pallas import tpu_sc as plsc

B = 16384
BPW = 512
D = 16
V_CLS = 1000
V_CLI = 100000


def _build():
    info = plsc.get_sparse_core_info()
    num_workers = info.num_cores * info.num_subcores
    assert B % num_workers == 0 and B // num_workers == BPW
    mesh = plsc.VectorSubcoreMesh(core_axis_name="c", subcore_axis_name="s")

    @functools.partial(
        pl.kernel,
        mesh=mesh,
        compiler_params=pltpu.CompilerParams(use_tc_tiling_on_sc=False,
                                             needs_layout_passes=False),
        out_type=jax.ShapeDtypeStruct((2 * D * B,), jnp.float32),
        scratch_types=[
            pltpu.VMEM((BPW,), jnp.int32),
            pltpu.VMEM((BPW,), jnp.int32),
            pltpu.VMEM((BPW, D), jnp.float32),
            pltpu.VMEM((BPW, D), jnp.float32),
            pltpu.VMEM((2 * D * BPW,), jnp.float32),
            pltpu.SemaphoreType.DMA,
            pltpu.SemaphoreType.DMA,
        ],
    )
    def cond_embed(y_cls_hbm, y_cli_hbm, e_cls_hbm, e_cli_hbm, out1d,
                   idx1_v, idx2_v, rows1_v, rows2_v, ob1d, sem1, sem2):
        wid = lax.axis_index("s") * info.num_cores + lax.axis_index("c")
        base = pl.multiple_of(wid * BPW, 128)
        iota = lax.iota(jnp.int32, 16)
        pltpu.sync_copy(y_cls_hbm.at[pl.ds(base, BPW)], idx1_v)
        pltpu.sync_copy(y_cli_hbm.at[pl.ds(base, BPW)], idx2_v)
        g1 = pltpu.async_copy(e_cls_hbm.at[idx1_v], rows1_v, sem1)
        g2 = pltpu.async_copy(e_cli_hbm.at[idx2_v], rows2_v, sem2)
        g1.wait()
        g2.wait()

        p_cls = (iota >> 3) * 4096 + (iota & 7) * 128
        p_cli = p_cls + 2 * 4096

        def cbody(g, _):
            for l in range(16):
                i = g * 16 + l
                off = (i >> 7) * 1024 + (i & 127)
                plsc.store_scatter(ob1d, [p_cls + off], rows1_v[i, :])
                plsc.store_scatter(ob1d, [p_cli + off], rows2_v[i, :])
            return 0

        lax.fori_loop(0, BPW // 16, cbody, 0)

        cb0 = base >> 7
        for dr in range(4):
            dst = pl.multiple_of((dr * 128 + cb0) * 1024, 1024)
            pltpu.sync_copy(ob1d.at[pl.ds(dr * 4096, 4096)],
                            out1d.at[pl.ds(dst, 4096)])

    return cond_embed


@jax.jit
def _run(y_cls, y_cli, e_cls, e_cli):
    out1d = _build()(y_cls.astype(jnp.int32), y_cli.astype(jnp.int32),
                     e_cls, e_cli)
    return out1d.reshape(4, 128, 8, 128).transpose(1, 3, 0, 2).reshape(
        B, 2 * D)


def kernel(y_cls, y_cli, e_cls, e_cli):
    assert y_cls.shape == (B,) and e_cls.shape == (V_CLS, D)
    assert y_cli.shape == (B,) and e_cli.shape == (V_CLI, D)
    return _run(y_cls, y_cli, e_cls, e_cli)
